# Initial kernel scaffold; baseline (speedup 1.0000x reference)
import jax
import jax.numpy as jnp
from jax.experimental import pallas as pl


def kernel(x0, x1, bbox0, bbox1):
    raise NotImplementedError("write your pallas kernel here")


# speedup vs baseline: 18.5853x; 18.5853x over previous
import jax
import jax.numpy as jnp
from jax import lax
from jax.experimental import pallas as pl
from jax.experimental.pallas import tpu as pltpu
from jax.experimental.pallas import tpu_sc as plsc

_OH, _OW = 7, 7
_NBIN = _OH * _OW
_NIDX = 2 * _NBIN
_C = 64

_L0H, _L0W = 200, 176
_L1H, _L1W = 100, 88
_P0 = _L0H * _L0W + _L0W + 1
_P1 = _L1H * _L1W + _L1W + 1
_NT = _P0 + _P1

_NC, _NS = 2, 16
_NW = _NC * _NS
_NROI = 2048
_RPW = _NROI // _NW

_LVLS = ((_L0H, _L0W, 1.0, 0), (_L1H, _L1W, 0.5, _P0))

_PI2_HI = 1.5707963705062866
_PI2_LO = -4.371139000186241e-08
_TWO_OVER_PI = 0.6366197723675814


def _cos_sin(t):
    kf0 = t * _TWO_OVER_PI
    ki = (kf0 + 0.5 * jnp.sign(kf0)).astype(jnp.int32)
    kf = ki.astype(jnp.float32)
    r = (t - kf * _PI2_HI) - kf * _PI2_LO
    z = r * r
    sn = ((-1.9515295891e-4 * z + 8.3321608736e-3) * z
          - 1.6666654611e-1) * z * r + r
    cs = ((2.443315711809948e-5 * z - 1.388731625493765e-3) * z
          + 4.166664568298827e-2) * z * z - 0.5 * z + 1.0
    q = jnp.bitwise_and(ki, 3)
    swap = jnp.bitwise_and(ki, 1) == 1
    cos_v = jnp.where(swap, sn, cs) * jnp.where((q == 1) | (q == 2), -1.0, 1.0)
    sin_v = jnp.where(swap, cs, sn) * jnp.where(q >= 2, -1.0, 1.0)
    return cos_v, sin_v


def _sc_body(tbl, bb0, bb1, out, bbv, prm, wref, idxr, rows, stage, sem):
    wid = lax.axis_index("s") * _NC + lax.axis_index("c")
    iota = lax.iota(jnp.int32, 16)
    iota49 = iota * _NBIN

    for lvl, (H, W, scale, base_off) in enumerate(_LVLS):
        bbh = bb0 if lvl == 0 else bb1
        for j, col in enumerate((0, 1, 3, 4, 6)):
            pltpu.sync_copy(bbh.at[col, pl.ds(wid * _RPW, _RPW)], bbv.at[j])

        for g in range(4):
            sl = pl.ds(g * 16, 16)
            cth, sth = _cos_sin(bbv[4, sl])
            prm[pl.ds(g * 16, 16)] = (bbv[0, sl] * (175.0 / 70.4) + 0.5) * scale
            prm[pl.ds(_RPW + g * 16, 16)] = (
                (bbv[1, sl] + 40.0) * (199.0 / 80.0) + 0.5) * scale
            prm[pl.ds(2 * _RPW + g * 16, 16)] = bbv[2, sl] * (175.0 / 70.4) * scale
            prm[pl.ds(3 * _RPW + g * 16, 16)] = bbv[3, sl] * (199.0 / 80.0) * scale
            prm[pl.ds(4 * _RPW + g * 16, 16)] = cth
            prm[pl.ds(5 * _RPW + g * 16, 16)] = sth

        def roi_body(r, carry, H=H, W=W, base_off=base_off, lvl=lvl):
            rv = jnp.full((16,), r, jnp.int32)
            cx = plsc.load_gather(prm, [rv])
            cy = plsc.load_gather(prm, [rv + _RPW])
            ww = plsc.load_gather(prm, [rv + 2 * _RPW])
            hh = plsc.load_gather(prm, [rv + 3 * _RPW])
            cth = plsc.load_gather(prm, [rv + 4 * _RPW])
            sth = plsc.load_gather(prm, [rv + 5 * _RPW])
            for g in range(4):
                b16 = g * 16 + iota
                ii = b16 // _OW
                jj = b16 - ii * _OW
                yl = ((ii.astype(jnp.float32) + 0.5) * (1.0 / _OH) - 0.5) * hh
                xl = ((jj.astype(jnp.float32) + 0.5) * (1.0 / _OW) - 0.5) * ww
                xs = cx + xl * cth - yl * sth
                ys = cy + xl * sth + yl * cth
                valid = ((ys > -1.0) & (ys < float(H))
                         & (xs > -1.0) & (xs < float(W)))
                yc = jnp.clip(ys, 0.0, H - 1.0)
                xc = jnp.clip(xs, 0.0, W - 1.0)
                y0 = yc.astype(jnp.int32)
                x0 = xc.astype(jnp.int32)
                ly = yc - y0.astype(jnp.float32)
                lx = xc - x0.astype(jnp.float32)
                hy = 1.0 - ly
                hx = 1.0 - lx
                vf = jnp.where(valid, 1.0, 0.0)
                wref[pl.ds(g * 16, 16)] = hy * hx * vf
                wref[pl.ds(64 + g * 16, 16)] = hy * lx * vf
                wref[pl.ds(128 + g * 16, 16)] = ly * hx * vf
                wref[pl.ds(192 + g * 16, 16)] = ly * lx * vf
                base = y0 * W + x0 + base_off
                m = b16 < _NBIN
                plsc.store_scatter(idxr, [b16 * 2], base, mask=m)
                plsc.store_scatter(idxr, [b16 * 2 + 1], base + W, mask=m)

            pltpu.async_copy(tbl.at[idxr], rows, sem).wait()

            def bin_body(b, c2):
                bv = jnp.full((16,), b, jnp.int32)
                w00 = plsc.load_gather(wref, [bv])
                w01 = plsc.load_gather(wref, [bv + 64])
                w10 = plsc.load_gather(wref, [bv + 128])
                w11 = plsc.load_gather(wref, [bv + 192])
                r0 = 2 * b
                r1 = r0 + 1
                for c in range(4):
                    v00 = rows[r0, pl.ds(c * 16, 16)]
                    v01 = rows[r0, pl.ds(_C + c * 16, 16)]
                    v10 = rows[r1, pl.ds(c * 16, 16)]
                    v11 = rows[r1, pl.ds(_C + c * 16, 16)]
                    acc = v00 * w00 + v01 * w01 + v10 * w10 + v11 * w11
                    plsc.store_scatter(stage, [iota49 + (c * 16 * _NBIN + b)],
                                       acc)
                return c2

            lax.fori_loop(0, _NBIN, bin_body, 0)
            gr = lvl * _NROI + wid * _RPW + r
            pltpu.sync_copy(stage, out.at[gr])
            return carry

        lax.fori_loop(0, _RPW, roi_body, 0)


def _build_table(x, pad_rows):
    c, h, w = x.shape[1], x.shape[2], x.shape[3]
    flat = jnp.transpose(x[0], (1, 2, 0)).reshape(h * w, c)
    flat = jnp.concatenate(
        [flat, jnp.zeros((pad_rows + 1 - h * w, c), jnp.float32)], axis=0)
    return jnp.concatenate([flat[:pad_rows], flat[1:pad_rows + 1]], axis=1)


def kernel(x0, x1, bbox0, bbox1):
    tbl = jnp.concatenate(
        [_build_table(x0, _P0), _build_table(x1, _P1)], axis=0)
    b0t = jnp.transpose(bbox0)
    b1t = jnp.transpose(bbox1)

    mesh = plsc.VectorSubcoreMesh(core_axis_name="c", subcore_axis_name="s")
    fn = pl.kernel(
        _sc_body,
        mesh=mesh,
        compiler_params=pltpu.CompilerParams(needs_layout_passes=False),
        out_type=jax.ShapeDtypeStruct((2 * _NROI, _C * _NBIN), jnp.float32),
        scratch_types=[
            pltpu.VMEM((5, _RPW), jnp.float32),
            pltpu.VMEM((6 * _RPW,), jnp.float32),
            pltpu.VMEM((4 * 64,), jnp.float32),
            pltpu.VMEM((_NIDX,), jnp.int32),
            pltpu.VMEM((_NIDX, 2 * _C), jnp.float32),
            pltpu.VMEM((_C * _NBIN,), jnp.float32),
            pltpu.SemaphoreType.DMA,
        ],
    )
    out = fn(tbl, b0t, b1t)
    return out.reshape(2 * _NROI, _C, _OH, _OW)


# speedup vs baseline: 21.9514x; 1.1811x over previous
import jax
import jax.numpy as jnp
from jax import lax
from jax.experimental import pallas as pl
from jax.experimental.pallas import tpu as pltpu
from jax.experimental.pallas import tpu_sc as plsc

_OH, _OW = 7, 7
_NBIN = _OH * _OW
_NIDX = 2 * _NBIN
_C = 64

_L0H, _L0W = 200, 176
_L1H, _L1W = 100, 88
_P0 = _L0H * _L0W + _L0W + 1
_P1 = _L1H * _L1W + _L1W + 1
_NT = _P0 + _P1

_NC, _NS = 2, 16
_NW = _NC * _NS
_NROI = 2048
_RPW = _NROI // _NW

_LVLS = ((_L0H, _L0W, 1.0, 0), (_L1H, _L1W, 0.5, _P0))

_PI2_HI = 1.5707963705062866
_PI2_LO = -4.371139000186241e-08
_TWO_OVER_PI = 0.6366197723675814


def _cos_sin(t):
    kf0 = t * _TWO_OVER_PI
    ki = (kf0 + 0.5 * jnp.sign(kf0)).astype(jnp.int32)
    kf = ki.astype(jnp.float32)
    r = (t - kf * _PI2_HI) - kf * _PI2_LO
    z = r * r
    sn = ((-1.9515295891e-4 * z + 8.3321608736e-3) * z
          - 1.6666654611e-1) * z * r + r
    cs = ((2.443315711809948e-5 * z - 1.388731625493765e-3) * z
          + 4.166664568298827e-2) * z * z - 0.5 * z + 1.0
    q = jnp.bitwise_and(ki, 3)
    swap = jnp.bitwise_and(ki, 1) == 1
    cos_v = jnp.where(swap, sn, cs) * jnp.where((q == 1) | (q == 2), -1.0, 1.0)
    sin_v = jnp.where(swap, cs, sn) * jnp.where(q >= 2, -1.0, 1.0)
    return cos_v, sin_v


def _sc_body(tbl, bb0, bb1, out,
             bbv, prm, wref0, wref1, idx0, idx1, rows0, rows1,
             stage0, stage1, gsem0, gsem1, osem0, osem1):
    wid = lax.axis_index("s") * _NC + lax.axis_index("c")
    iota = lax.iota(jnp.int32, 16)
    iota49 = iota * _NBIN

    for lvl, (H, W, scale, base_off) in enumerate(_LVLS):
        bbh = bb0 if lvl == 0 else bb1
        for j, col in enumerate((0, 1, 3, 4, 6)):
            pltpu.sync_copy(bbh.at[col, pl.ds(wid * _RPW, _RPW)], bbv.at[j])

        for g in range(4):
            sl = pl.ds(g * 16, 16)
            cth, sth = _cos_sin(bbv[4, sl])
            prm[pl.ds(g * 16, 16)] = (bbv[0, sl] * (175.0 / 70.4) + 0.5) * scale
            prm[pl.ds(_RPW + g * 16, 16)] = (
                (bbv[1, sl] + 40.0) * (199.0 / 80.0) + 0.5) * scale
            prm[pl.ds(2 * _RPW + g * 16, 16)] = bbv[2, sl] * (175.0 / 70.4) * scale
            prm[pl.ds(3 * _RPW + g * 16, 16)] = bbv[3, sl] * (199.0 / 80.0) * scale
            prm[pl.ds(4 * _RPW + g * 16, 16)] = cth
            prm[pl.ds(5 * _RPW + g * 16, 16)] = sth

        def fire(r, wref, idxr, rows, sem, H=H, W=W, base_off=base_off):
            rv = jnp.full((16,), r, jnp.int32)
            cx = plsc.load_gather(prm, [rv])
            cy = plsc.load_gather(prm, [rv + _RPW])
            ww = plsc.load_gather(prm, [rv + 2 * _RPW])
            hh = plsc.load_gather(prm, [rv + 3 * _RPW])
            cth = plsc.load_gather(prm, [rv + 4 * _RPW])
            sth = plsc.load_gather(prm, [rv + 5 * _RPW])
            for g in range(4):
                b16 = g * 16 + iota
                ii = b16 // _OW
                jj = b16 - ii * _OW
                yl = ((ii.astype(jnp.float32) + 0.5) * (1.0 / _OH) - 0.5) * hh
                xl = ((jj.astype(jnp.float32) + 0.5) * (1.0 / _OW) - 0.5) * ww
                xs = cx + xl * cth - yl * sth
                ys = cy + xl * sth + yl * cth
                valid = ((ys > -1.0) & (ys < float(H))
                         & (xs > -1.0) & (xs < float(W)))
                yc = jnp.clip(ys, 0.0, H - 1.0)
                xc = jnp.clip(xs, 0.0, W - 1.0)
                y0 = yc.astype(jnp.int32)
                x0 = xc.astype(jnp.int32)
                ly = yc - y0.astype(jnp.float32)
                lx = xc - x0.astype(jnp.float32)
                hy = 1.0 - ly
                hx = 1.0 - lx
                vf = jnp.where(valid, 1.0, 0.0)
                wref[pl.ds(g * 16, 16)] = hy * hx * vf
                wref[pl.ds(64 + g * 16, 16)] = hy * lx * vf
                wref[pl.ds(128 + g * 16, 16)] = ly * hx * vf
                wref[pl.ds(192 + g * 16, 16)] = ly * lx * vf
                base = y0 * W + x0 + base_off
                m = b16 < _NBIN
                plsc.store_scatter(idxr, [b16 * 2], base, mask=m)
                plsc.store_scatter(idxr, [b16 * 2 + 1], base + W, mask=m)
            pltpu.async_copy(tbl.at[idxr], rows, sem)

        def combine(r, wref, rows, stage, osem, lvl=lvl):
            def bin7(k7, c2):
                for u in range(7):
                    b = k7 * 7 + u
                    bv = jnp.full((16,), b, jnp.int32)
                    w00 = plsc.load_gather(wref, [bv])
                    w01 = plsc.load_gather(wref, [bv + 64])
                    w10 = plsc.load_gather(wref, [bv + 128])
                    w11 = plsc.load_gather(wref, [bv + 192])
                    r0 = 2 * b
                    r1 = r0 + 1
                    for c in range(4):
                        v00 = rows[r0, pl.ds(c * 16, 16)]
                        v01 = rows[r0, pl.ds(_C + c * 16, 16)]
                        v10 = rows[r1, pl.ds(c * 16, 16)]
                        v11 = rows[r1, pl.ds(_C + c * 16, 16)]
                        acc = v00 * w00 + v01 * w01 + v10 * w10 + v11 * w11
                        plsc.store_scatter(
                            stage, [iota49 + (c * 16 * _NBIN + b)], acc)
                return c2

            lax.fori_loop(0, 7, bin7, 0)
            gr = lvl * _NROI + wid * _RPW + r
            pltpu.async_copy(stage, out.at[gr], osem)

        def step(t, carry):
            r = 2 * t
            fire(r, wref0, idx0, rows0, gsem0)
            fire(r + 1, wref1, idx1, rows1, gsem1)
            pltpu.make_async_copy(tbl.at[idx0], rows0, gsem0).wait()

            @pl.when(t > 0)
            def _():
                pltpu.make_async_copy(stage0, out.at[0], osem0).wait()
            combine(r, wref0, rows0, stage0, osem0)
            pltpu.make_async_copy(tbl.at[idx1], rows1, gsem1).wait()

            @pl.when(t > 0)
            def _():
                pltpu.make_async_copy(stage1, out.at[0], osem1).wait()
            combine(r + 1, wref1, rows1, stage1, osem1)
            return carry

        lax.fori_loop(0, _RPW // 2, step, 0)
        pltpu.make_async_copy(stage0, out.at[0], osem0).wait()
        pltpu.make_async_copy(stage1, out.at[0], osem1).wait()


def _build_table(x, pad_rows):
    c, h, w = x.shape[1], x.shape[2], x.shape[3]
    flat = jnp.transpose(x[0], (1, 2, 0)).reshape(h * w, c)
    flat = jnp.concatenate(
        [flat, jnp.zeros((pad_rows + 1 - h * w, c), jnp.float32)], axis=0)
    return jnp.concatenate([flat[:pad_rows], flat[1:pad_rows + 1]], axis=1)


def kernel(x0, x1, bbox0, bbox1):
    tbl = jnp.concatenate(
        [_build_table(x0, _P0), _build_table(x1, _P1)], axis=0)
    b0t = jnp.transpose(bbox0)
    b1t = jnp.transpose(bbox1)

    mesh = plsc.VectorSubcoreMesh(core_axis_name="c", subcore_axis_name="s")
    fn = pl.kernel(
        _sc_body,
        mesh=mesh,
        compiler_params=pltpu.CompilerParams(needs_layout_passes=False),
        out_type=jax.ShapeDtypeStruct((2 * _NROI, _C * _NBIN), jnp.float32),
        scratch_types=[
            pltpu.VMEM((5, _RPW), jnp.float32),
            pltpu.VMEM((6 * _RPW,), jnp.float32),
            pltpu.VMEM((4 * 64,), jnp.float32),
            pltpu.VMEM((4 * 64,), jnp.float32),
            pltpu.VMEM((_NIDX,), jnp.int32),
            pltpu.VMEM((_NIDX,), jnp.int32),
            pltpu.VMEM((_NIDX, 2 * _C), jnp.float32),
            pltpu.VMEM((_NIDX, 2 * _C), jnp.float32),
            pltpu.VMEM((_C * _NBIN,), jnp.float32),
            pltpu.VMEM((_C * _NBIN,), jnp.float32),
            pltpu.SemaphoreType.DMA,
            pltpu.SemaphoreType.DMA,
            pltpu.SemaphoreType.DMA,
            pltpu.SemaphoreType.DMA,
        ],
    )
    out = fn(tbl, b0t, b1t)
    return out.reshape(2 * _NROI, _C, _OH, _OW)
